# baseline (speedup 1.0000x reference)
import functools

import jax
import jax.numpy as jnp
from jax import lax
from jax.experimental import pallas as pl
from jax.experimental.pallas import tpu as pltpu
from jax.experimental.pallas import tpu_sc as plsc

BS = 32
SEQ = 64
D = 128
NQ = 50000
NC = 1000
NEG = -1e30
NPOS = BS * SEQ
CN_PAD = 16384
HR = 72


def _sc_gather(eq_tab, qn_tab, q2c_tab, qseq_idx, qa_idx, qb_idx, cn_idx2):
    mesh = plsc.VectorSubcoreMesh(core_axis_name="c", subcore_axis_name="s")

    @functools.partial(
        pl.kernel,
        mesh=mesh,
        out_type=[
            jax.ShapeDtypeStruct((CN_PAD, D), jnp.float32),
            jax.ShapeDtypeStruct((NPOS, D), jnp.float32),
            jax.ShapeDtypeStruct((NPOS, 128), jnp.int32),
            jax.ShapeDtypeStruct((NPOS, 128), jnp.int32),
        ],
        scratch_types=[
            pltpu.VMEM((4, 128), jnp.int32),
            pltpu.VMEM((64,), jnp.int32),
            pltpu.VMEM((64,), jnp.int32),
            pltpu.VMEM((64,), jnp.int32),
            pltpu.VMEM((512, D), jnp.float32),
            pltpu.VMEM((64, D), jnp.float32),
            pltpu.VMEM((64, 128), jnp.int32),
            pltpu.VMEM((64, 128), jnp.int32),
            pltpu.SemaphoreType.DMA,
        ],
    )
    def k(eq_hbm, qn_hbm, q2c_hbm, qidx_hbm, qaidx_hbm, qbidx_hbm, cnidx_hbm,
          cn_out, eq_out, qn_out, q2c_out,
          cnidx_v, qidx_v, qaidx_v, qbidx_v, cnrows_v,
          eqrows_v, qnrows_v, q2crows_v, sem):
        wid = lax.axis_index("s") * 2 + lax.axis_index("c")
        pltpu.sync_copy(cnidx_hbm.at[pl.ds(wid * 4, 4)], cnidx_v)
        pltpu.sync_copy(qidx_hbm.at[pl.ds(wid * 64, 64)], qidx_v)
        pltpu.sync_copy(qaidx_hbm.at[pl.ds(wid * 64, 64)], qaidx_v)
        pltpu.sync_copy(qbidx_hbm.at[pl.ds(wid * 64, 64)], qbidx_v)
        cps = []
        for c in range(4):
            cps.append(pltpu.async_copy(
                eq_hbm.at[cnidx_v.at[c]],
                cnrows_v.at[pl.ds(c * 128, 128)], sem))
        cps.append(pltpu.async_copy(eq_hbm.at[qidx_v], eqrows_v, sem))
        cps.append(pltpu.async_copy(qn_hbm.at[qaidx_v], qnrows_v, sem))
        cps.append(pltpu.async_copy(q2c_hbm.at[qbidx_v], q2crows_v, sem))
        for cp in cps:
            cp.wait()
        pltpu.sync_copy(cnrows_v, cn_out.at[pl.ds(wid * 512, 512)])
        pltpu.sync_copy(eqrows_v, eq_out.at[pl.ds(wid * 64, 64)])
        pltpu.sync_copy(qnrows_v, qn_out.at[pl.ds(wid * 64, 64)])
        pltpu.sync_copy(q2crows_v, q2c_out.at[pl.ds(wid * 64, 64)])

    return k(eq_tab, qn_tab, q2c_tab, qseq_idx, qa_idx, qb_idx, cn_idx2)


def _a1_body(cn3_ref, ec_ref, w1t_ref, b1_ref, out_ref):
    m = jnp.sum(cn3_ref[...], axis=1) * (1.0 / 16.0)
    ec = ec_ref[...]
    a1 = jnp.tanh((m + ec) @ w1t_ref[...] + b1_ref[...])
    out_ref[...] = jnp.concatenate([ec, a1], axis=1)


def _run_a1(cn3, ec, w1t, b1):
    return pl.pallas_call(
        _a1_body,
        grid=(5,),
        in_specs=[
            pl.BlockSpec((200, 16, D), lambda i: (i, 0, 0)),
            pl.BlockSpec((200, D), lambda i: (i, 0)),
            pl.BlockSpec((D, D), lambda i: (0, 0)),
            pl.BlockSpec((1, D), lambda i: (0, 0)),
        ],
        out_specs=pl.BlockSpec((200, 2 * D), lambda i: (i, 0)),
        out_shape=jax.ShapeDtypeStruct((NC, 2 * D), jnp.float32),
    )(cn3, ec, w1t, b1)


def _a2_body(qn16_ref, q2c4_ref, eq_ref, eqs_ref, ms_ref,
             rs_ref, ca_ref,
             w0t_ref, wlt_ref, b0_ref, bl_ref, wiat_ref, wibt_ref, bih_ref,
             er_ref, kw_ref, ww_ref, gi1_ref, qc8_ref):
    ints = qn16_ref[...]
    ints2 = q2c4_ref[...]
    iota_c = lax.broadcasted_iota(jnp.int32, (256, NC), 1)
    s = jnp.zeros((256, NC), jnp.float32)
    for j in range(16):
        s = s + (ints[:, j:j + 1] == iota_c).astype(jnp.float32)
    cam = (s @ ca_ref[...]) * (1.0 / 16.0)
    cmean = cam[:, :D]
    amean = cam[:, D:]
    eq = eq_ref[...]
    e0a = jnp.tanh((cmean + eq) @ w0t_ref[...] + b0_ref[...])
    e0b = jnp.tanh((amean + e0a) @ w0t_ref[...] + b0_ref[...])
    agg = jnp.tanh(e0b @ wlt_ref[...] + bl_ref[...])
    mf = ms_ref[...].astype(jnp.float32)
    emb_q = mf * agg + (1.0 - mf) * eq
    rf = rs_ref[...].astype(jnp.float32)
    er = er_ref[...]
    emb_r = rf * er[1:2, :] + (1.0 - rf) * er[0:1, :]
    gi1_ref[...] = emb_q @ wiat_ref[...] + emb_r @ wibt_ref[...] + bih_ref[...]
    qc8_ref[:, 0, :] = eqs_ref[...]
    ec = ca_ref[...][:, :D]
    for j in range(4):
        oh = (ints2[:, j:j + 1] == iota_c).astype(jnp.float32)
        qc8_ref[:, j + 1, :] = oh @ ec
    kv = ww_ref[...][:, D:] @ kw_ref[...]
    qc8_ref[:, 5, :] = jnp.broadcast_to(kv, (256, D))
    qc8_ref[:, 6, :] = jnp.zeros((256, D), jnp.float32)
    qc8_ref[:, 7, :] = jnp.zeros((256, D), jnp.float32)


def _run_a2(qn16, q2c4, eq_rows, eq_shift, ms, rs, ca,
            w0t, wlt, b0, bl, wiat, wibt, bih, er, key_W, W_W):
    return pl.pallas_call(
        _a2_body,
        grid=(8,),
        in_specs=[
            pl.BlockSpec((256, 16), lambda k: (k, 0)),
            pl.BlockSpec((256, 4), lambda k: (k, 0)),
            pl.BlockSpec((256, D), lambda k: (k, 0)),
            pl.BlockSpec((256, D), lambda k: (k, 0)),
            pl.BlockSpec((256, 1), lambda k: (k, 0)),
            pl.BlockSpec((256, 1), lambda k: (k, 0)),
            pl.BlockSpec((NC, 2 * D), lambda k: (0, 0)),
            pl.BlockSpec((D, D), lambda k: (0, 0)),
            pl.BlockSpec((D, D), lambda k: (0, 0)),
            pl.BlockSpec((1, D), lambda k: (0, 0)),
            pl.BlockSpec((1, D), lambda k: (0, 0)),
            pl.BlockSpec((D, 3 * D), lambda k: (0, 0)),
            pl.BlockSpec((D, 3 * D), lambda k: (0, 0)),
            pl.BlockSpec((1, 3 * D), lambda k: (0, 0)),
            pl.BlockSpec((2, D), lambda k: (0, 0)),
            pl.BlockSpec((D, D), lambda k: (0, 0)),
            pl.BlockSpec((1, 2 * D), lambda k: (0, 0)),
        ],
        out_specs=[
            pl.BlockSpec((256, 3 * D), lambda k: (k, 0)),
            pl.BlockSpec((256, 8, D), lambda k: (k, 0, 0)),
        ],
        out_shape=[
            jax.ShapeDtypeStruct((NPOS, 3 * D), jnp.float32),
            jax.ShapeDtypeStruct((NPOS, 8, D), jnp.float32),
        ],
    )(qn16, q2c4, eq_rows, eq_shift, ms, rs, ca,
      w0t, wlt, b0, bl, wiat, wibt, bih, er, key_W, W_W)


def _a3_body(eqb_ref, mask_ref):
    eq = eqb_ref[...]
    sh = jnp.concatenate([eq[:, 1:], eq[:, :1]], axis=1)
    smat = lax.dot_general(sh, eq, (((2,), (2,)), ((0,), (0,))))
    tg = lax.broadcasted_iota(jnp.int32, (BS, SEQ, SEQ), 1)
    taug = lax.broadcasted_iota(jnp.int32, (BS, SEQ, SEQ), 2)
    sc = jnp.where(taug < tg, smat, NEG)
    nsel = jnp.minimum(tg[:, :, :1], 10)
    sel = taug == tg
    for p in range(10):
        m = jnp.max(sc, axis=2, keepdims=True)
        cand = jnp.where(sc == m, taug, 9999)
        idx = jnp.min(cand, axis=2, keepdims=True)
        pick = jnp.logical_and(taug == idx, p < nsel)
        sel = jnp.logical_or(sel, pick)
        sc = jnp.where(pick, NEG, sc)
    mask_ref[...] = jnp.swapaxes(jnp.where(sel, 0.0, NEG), 0, 1)


def _run_a3(eq_b):
    return pl.pallas_call(
        _a3_body,
        grid=(1,),
        in_specs=[pl.BlockSpec((BS, SEQ, D), lambda i: (0, 0, 0))],
        out_specs=pl.BlockSpec((SEQ, BS, SEQ), lambda i: (0, 0, 0)),
        out_shape=jax.ShapeDtypeStruct((SEQ, BS, SEQ), jnp.float32),
    )(eq_b)


def _b_body(gi1_ref, qc8_ref, mask_ref, h1i_ref, h2i_ref,
            w1hh_ref, w2ih_ref, w2hh_ref, b1hh_ref, b2ih_ref, b2hh_ref,
            qw_ref, qb_ref, kb_ref, ww_ref, wb_ref,
            y_ref, h1_s, h2_s, hist_s):
    u = pl.program_id(0)
    ww = ww_ref[...]
    w1v = ww[:, :D]
    w2v = ww[:, D:]

    @pl.when(u == 0)
    def _init():
        h1_s[...] = h1i_ref[...]
        h2_s[...] = h2i_ref[...]
        hist_s[...] = jnp.zeros_like(hist_s)
        qv = w1v @ qw_ref[...]
        hist_s[:, SEQ:SEQ + 1, :] = jnp.broadcast_to(qv[None], (BS, 1, D))
        y_ref[...] = jnp.zeros_like(y_ref)

    @pl.when(u == 2)
    def _clear_row0():
        hist_s[:, 0:1, :] = jnp.zeros((BS, 1, D), jnp.float32)

    hist = hist_s[...]

    qb1 = jnp.sum(qb_ref[...] * w1v, axis=1, keepdims=True)
    kb2 = (jnp.sum(kb_ref[...] * w2v, axis=1, keepdims=True)
           + wb_ref[...][:, :1])
    qc8 = qc8_ref[0]
    d = lax.dot_general(qc8, hist, (((2,), (2,)), ((0,), (0,))))
    g = jax.nn.sigmoid(d[:, :5, :SEQ])
    qw1 = d[:, :5, SEQ:SEQ + 1]
    kw2 = d[:, 5:6, :SEQ]
    mt = mask_ref[0][:, None, :]
    w = qw1 + kw2 + mt + jnp.reshape(qb1 + kb2, (1, 1, 1))
    m = jnp.max(jnp.max(w, axis=2, keepdims=True), axis=1, keepdims=True)
    e = jnp.exp(w - m)
    num = jnp.sum(jnp.sum(e * g, axis=2, keepdims=True), axis=1)
    den = jnp.sum(jnp.sum(e, axis=2, keepdims=True), axis=1)
    yt = num / den

    @pl.when(u > 0)
    def _ywrite():
        col = jnp.where(u == 1, 0, u)
        lane = lax.broadcasted_iota(jnp.int32, (BS, SEQ), 1)
        y_ref[...] = jnp.where(lane == col, jnp.broadcast_to(yt, (BS, SEQ)),
                               y_ref[...])

    h1 = h1_s[...]
    h2 = h2_s[...]
    gi1 = gi1_ref[0]
    gh1 = h1 @ w1hh_ref[...] + b1hh_ref[...]
    r1 = jax.nn.sigmoid(gi1[:, :D] + gh1[:, :D])
    z1 = jax.nn.sigmoid(gi1[:, D:2 * D] + gh1[:, D:2 * D])
    n1 = jnp.tanh(gi1[:, 2 * D:] + r1 * gh1[:, 2 * D:])
    h1n = (1.0 - z1) * n1 + z1 * h1
    gi2 = h1n @ w2ih_ref[...] + b2ih_ref[...]
    gh2 = h2 @ w2hh_ref[...] + b2hh_ref[...]
    r2 = jax.nn.sigmoid(gi2[:, :D] + gh2[:, :D])
    z2 = jax.nn.sigmoid(gi2[:, D:2 * D] + gh2[:, D:2 * D])
    n2 = jnp.tanh(gi2[:, 2 * D:] + r2 * gh2[:, 2 * D:])
    g2 = (1.0 - z2) * n2 + z2 * h2

    @pl.when(u < SEQ - 1)
    def _state_upd():
        hist_s[:, pl.ds(u, 1), :] = g2[:, None, :]
        h1_s[...] = h1n

    @pl.when(jnp.logical_and(u >= 1, u < SEQ - 1))
    def _h2_upd():
        h2_s[...] = g2


def _run_b(gi1r, qc8r, mask_t, h1_init, h2_init,
           w1hh_t, w2ih_t, w2hh_t, b1hh, b2ih, b2hh,
           query_W, query_b, key_b, W_W, wb):
    return pl.pallas_call(
        _b_body,
        grid=(SEQ,),
        in_specs=[
            pl.BlockSpec((1, BS, 3 * D), lambda u: (jnp.minimum(u, SEQ - 2), 0, 0)),
            pl.BlockSpec((1, BS, 8, D), lambda u: (jnp.maximum(u - 1, 0), 0, 0, 0)),
            pl.BlockSpec((1, BS, SEQ), lambda u: (jnp.maximum(u - 1, 0), 0, 0)),
            pl.BlockSpec((BS, D), lambda t: (0, 0)),
            pl.BlockSpec((BS, D), lambda t: (0, 0)),
            pl.BlockSpec((D, 3 * D), lambda t: (0, 0)),
            pl.BlockSpec((D, 3 * D), lambda t: (0, 0)),
            pl.BlockSpec((D, 3 * D), lambda t: (0, 0)),
            pl.BlockSpec((1, 3 * D), lambda t: (0, 0)),
            pl.BlockSpec((1, 3 * D), lambda t: (0, 0)),
            pl.BlockSpec((1, 3 * D), lambda t: (0, 0)),
            pl.BlockSpec((D, D), lambda t: (0, 0)),
            pl.BlockSpec((1, D), lambda t: (0, 0)),
            pl.BlockSpec((1, D), lambda t: (0, 0)),
            pl.BlockSpec((1, 2 * D), lambda t: (0, 0)),
            pl.BlockSpec((1, D), lambda t: (0, 0)),
        ],
        out_specs=pl.BlockSpec((BS, SEQ), lambda t: (0, 0)),
        out_shape=jax.ShapeDtypeStruct((BS, SEQ), jnp.float32),
        scratch_shapes=[
            pltpu.VMEM((BS, D), jnp.float32),
            pltpu.VMEM((BS, D), jnp.float32),
            pltpu.VMEM((BS, HR, D), jnp.float32),
        ],
    )(gi1r, qc8r, mask_t, h1_init, h2_init,
      w1hh_t, w2ih_t, w2hh_t, b1hh, b2ih, b2hh,
      query_W, query_b, key_b, W_W, wb)


def kernel(question_seq, correct_seq, mask_seq, question_neighbors,
           concept_neighbors, q2c, embed_question, embed_concept,
           embed_correct, gru1_W_ih, gru1_W_hh, gru1_b_ih, gru1_b_hh,
           gru2_W_ih, gru2_W_hh, gru2_b_ih, gru2_b_hh,
           agg_W0, agg_b0, agg_W1, agg_b1, agg_last_W, agg_last_b,
           query_W, query_b, key_W, key_b, W_W, W_b, h1_init, h2_init):
    f32 = jnp.float32
    qseq_t = question_seq.astype(jnp.int32).T.reshape(-1)
    qnext_t = jnp.concatenate([qseq_t[BS:], qseq_t[:BS]])
    cn_idx2 = jnp.pad(concept_neighbors.astype(jnp.int32).reshape(-1),
                      (0, CN_PAD - NC * 16)).reshape(128, 128)
    qn_tab = question_neighbors.astype(jnp.int32).reshape(6250, 128)
    q2c_tab = jnp.pad(q2c.astype(jnp.int32).reshape(-1),
                      (0, 64)).reshape(1563, 128)

    cn_rows, eq_rows, qnw, q2cw = _sc_gather(
        embed_question.astype(f32), qn_tab, q2c_tab,
        qseq_t, qseq_t >> 3, qnext_t >> 5, cn_idx2)

    cn3 = cn_rows[:NC * 16].reshape(NC, 16, D)
    ca = _run_a1(cn3, embed_concept.astype(f32), agg_W1.T,
                 agg_b1.reshape(1, D))

    eq_shift = jnp.concatenate([eq_rows[BS:], eq_rows[:BS]], axis=0)
    ms = mask_seq.astype(jnp.int32).T.reshape(NPOS, 1)
    rs = correct_seq.astype(jnp.int32).T.reshape(NPOS, 1)
    qn16 = jnp.take_along_axis(
        qnw.reshape(NPOS, 8, 16),
        (qseq_t & 7).reshape(NPOS, 1, 1), axis=1)[:, 0, :]
    q2c4 = jnp.take_along_axis(
        q2cw.reshape(NPOS, 32, 4),
        (qnext_t & 31).reshape(NPOS, 1, 1), axis=1)[:, 0, :]
    gi1, qc8 = _run_a2(
        qn16, q2c4, eq_rows, eq_shift, ms, rs, ca,
        agg_W0.T, agg_last_W.T, agg_b0.reshape(1, D),
        agg_last_b.reshape(1, D),
        gru1_W_ih[:, :D].T, gru1_W_ih[:, D:].T,
        gru1_b_ih.reshape(1, 3 * D), embed_correct.astype(f32),
        key_W.astype(f32), W_W.reshape(1, 2 * D))

    eq_b = eq_rows.reshape(SEQ, BS, D).transpose(1, 0, 2)
    mask_t = _run_a3(eq_b)

    gi1r = gi1.reshape(SEQ, BS, 3 * D)
    qc8r = qc8.reshape(SEQ, BS, 8, D)
    y = _run_b(
        gi1r, qc8r, mask_t, h1_init.astype(f32), h2_init.astype(f32),
        gru1_W_hh.T, gru2_W_ih.T, gru2_W_hh.T,
        gru1_b_hh.reshape(1, 3 * D), gru2_b_ih.reshape(1, 3 * D),
        gru2_b_hh.reshape(1, 3 * D),
        query_W.astype(f32), query_b.reshape(1, D), key_b.reshape(1, D),
        W_W.reshape(1, 2 * D),
        jnp.broadcast_to(W_b.reshape(1, 1), (1, D)))
    return y


# speedup vs baseline: 1.3692x; 1.3692x over previous
import functools

import jax
import jax.numpy as jnp
from jax import lax
from jax.experimental import pallas as pl
from jax.experimental.pallas import tpu as pltpu
from jax.experimental.pallas import tpu_sc as plsc

BS = 32
SEQ = 64
D = 128
NQ = 50000
NC = 1000
NEG = -1e30
NPOS = BS * SEQ
CN_PAD = 16384
HR = 72


def _sc_gather(eq_tab, int_tab, qseq_idx, cn_idx2):
    mesh = plsc.VectorSubcoreMesh(core_axis_name="c", subcore_axis_name="s")

    @functools.partial(
        pl.kernel,
        mesh=mesh,
        out_type=[
            jax.ShapeDtypeStruct((CN_PAD, D), jnp.float32),
            jax.ShapeDtypeStruct((NPOS, D), jnp.float32),
            jax.ShapeDtypeStruct((NPOS, 128), jnp.int32),
        ],
        scratch_types=[
            pltpu.VMEM((4, 128), jnp.int32),
            pltpu.VMEM((64,), jnp.int32),
            pltpu.VMEM((512, D), jnp.float32),
            pltpu.VMEM((64, D), jnp.float32),
            pltpu.VMEM((64, 128), jnp.int32),
            pltpu.SemaphoreType.DMA,
        ],
    )
    def k(eq_hbm, int_hbm, qidx_hbm, cnidx_hbm, cn_out, eq_out, int_out,
          cnidx_v, qidx_v, cnrows_v, eqrows_v, introws_v, sem):
        wid = lax.axis_index("s") * 2 + lax.axis_index("c")
        pltpu.sync_copy(cnidx_hbm.at[pl.ds(wid * 4, 4)], cnidx_v)
        pltpu.sync_copy(qidx_hbm.at[pl.ds(wid * 64, 64)], qidx_v)
        cps = []
        for c in range(4):
            cps.append(pltpu.async_copy(
                eq_hbm.at[cnidx_v.at[c]],
                cnrows_v.at[pl.ds(c * 128, 128)], sem))
        cps.append(pltpu.async_copy(eq_hbm.at[qidx_v], eqrows_v, sem))
        cps.append(pltpu.async_copy(int_hbm.at[qidx_v], introws_v, sem))
        for cp in cps:
            cp.wait()
        pltpu.sync_copy(cnrows_v, cn_out.at[pl.ds(wid * 512, 512)])
        pltpu.sync_copy(eqrows_v, eq_out.at[pl.ds(wid * 64, 64)])
        pltpu.sync_copy(introws_v, int_out.at[pl.ds(wid * 64, 64)])

    return k(eq_tab, int_tab, qseq_idx, cn_idx2)


def _a1_body(cn3_ref, ec_ref, w1t_ref, b1_ref, out_ref):
    m = jnp.sum(cn3_ref[...], axis=1) * (1.0 / 16.0)
    ec = ec_ref[...]
    a1 = jnp.tanh((m + ec) @ w1t_ref[...] + b1_ref[...])
    out_ref[...] = jnp.concatenate([ec, a1], axis=1)


def _run_a1(cn3, ec, w1t, b1):
    return pl.pallas_call(
        _a1_body,
        grid=(5,),
        in_specs=[
            pl.BlockSpec((200, 16, D), lambda i: (i, 0, 0)),
            pl.BlockSpec((200, D), lambda i: (i, 0)),
            pl.BlockSpec((D, D), lambda i: (0, 0)),
            pl.BlockSpec((1, D), lambda i: (0, 0)),
        ],
        out_specs=pl.BlockSpec((200, 2 * D), lambda i: (i, 0)),
        out_shape=jax.ShapeDtypeStruct((NC, 2 * D), jnp.float32),
    )(cn3, ec, w1t, b1)


def _a2_body(int_ref, ints_ref, eq_ref, eqs_ref, ms_ref,
             rs_ref, ca_ref,
             w0t_ref, wlt_ref, b0_ref, bl_ref, wiat_ref, wibt_ref, bih_ref,
             er_ref, kw_ref, ww_ref, gi1_ref, qc8_ref):
    ints = int_ref[...]
    ints2 = ints_ref[...]
    iota_c = lax.broadcasted_iota(jnp.int32, (256, NC), 1)
    s = jnp.zeros((256, NC), jnp.float32)
    for j in range(16):
        s = s + (ints[:, j:j + 1] == iota_c).astype(jnp.float32)
    cam = (s @ ca_ref[...]) * (1.0 / 16.0)
    cmean = cam[:, :D]
    amean = cam[:, D:]
    eq = eq_ref[...]
    e0a = jnp.tanh((cmean + eq) @ w0t_ref[...] + b0_ref[...])
    e0b = jnp.tanh((amean + e0a) @ w0t_ref[...] + b0_ref[...])
    agg = jnp.tanh(e0b @ wlt_ref[...] + bl_ref[...])
    mf = ms_ref[...].astype(jnp.float32)
    emb_q = mf * agg + (1.0 - mf) * eq
    rf = rs_ref[...].astype(jnp.float32)
    er = er_ref[...]
    emb_r = rf * er[1:2, :] + (1.0 - rf) * er[0:1, :]
    gi1_ref[...] = emb_q @ wiat_ref[...] + emb_r @ wibt_ref[...] + bih_ref[...]
    qc8_ref[:, 0, :] = eqs_ref[...]
    ec = ca_ref[...][:, :D]
    for j in range(4):
        oh = (ints2[:, 16 + j:17 + j] == iota_c).astype(jnp.float32)
        qc8_ref[:, j + 1, :] = oh @ ec
    kv = ww_ref[...][:, D:] @ kw_ref[...]
    qc8_ref[:, 5, :] = jnp.broadcast_to(kv, (256, D))
    qc8_ref[:, 6, :] = jnp.zeros((256, D), jnp.float32)
    qc8_ref[:, 7, :] = jnp.zeros((256, D), jnp.float32)


def _run_a2(int_rows, int_shift, eq_rows, eq_shift, ms, rs, ca,
            w0t, wlt, b0, bl, wiat, wibt, bih, er, key_W, W_W):
    return pl.pallas_call(
        _a2_body,
        grid=(8,),
        in_specs=[
            pl.BlockSpec((256, 128), lambda k: (k, 0)),
            pl.BlockSpec((256, 128), lambda k: (k, 0)),
            pl.BlockSpec((256, D), lambda k: (k, 0)),
            pl.BlockSpec((256, D), lambda k: (k, 0)),
            pl.BlockSpec((256, 1), lambda k: (k, 0)),
            pl.BlockSpec((256, 1), lambda k: (k, 0)),
            pl.BlockSpec((NC, 2 * D), lambda k: (0, 0)),
            pl.BlockSpec((D, D), lambda k: (0, 0)),
            pl.BlockSpec((D, D), lambda k: (0, 0)),
            pl.BlockSpec((1, D), lambda k: (0, 0)),
            pl.BlockSpec((1, D), lambda k: (0, 0)),
            pl.BlockSpec((D, 3 * D), lambda k: (0, 0)),
            pl.BlockSpec((D, 3 * D), lambda k: (0, 0)),
            pl.BlockSpec((1, 3 * D), lambda k: (0, 0)),
            pl.BlockSpec((2, D), lambda k: (0, 0)),
            pl.BlockSpec((D, D), lambda k: (0, 0)),
            pl.BlockSpec((1, 2 * D), lambda k: (0, 0)),
        ],
        out_specs=[
            pl.BlockSpec((256, 3 * D), lambda k: (k, 0)),
            pl.BlockSpec((256, 8, D), lambda k: (k, 0, 0)),
        ],
        out_shape=[
            jax.ShapeDtypeStruct((NPOS, 3 * D), jnp.float32),
            jax.ShapeDtypeStruct((NPOS, 8, D), jnp.float32),
        ],
    )(int_rows, int_shift, eq_rows, eq_shift, ms, rs, ca,
      w0t, wlt, b0, bl, wiat, wibt, bih, er, key_W, W_W)


def _a3_body(eqb_ref, mask_ref):
    eq = eqb_ref[...]
    sh = jnp.concatenate([eq[:, 1:], eq[:, :1]], axis=1)
    smat = lax.dot_general(sh, eq, (((2,), (2,)), ((0,), (0,))))
    tg = lax.broadcasted_iota(jnp.int32, (BS, SEQ, SEQ), 1)
    taug = lax.broadcasted_iota(jnp.int32, (BS, SEQ, SEQ), 2)
    sc = jnp.where(taug < tg, smat, NEG)
    nsel = jnp.minimum(tg[:, :, :1], 10)
    sel = taug == tg
    for p in range(10):
        m = jnp.max(sc, axis=2, keepdims=True)
        cand = jnp.where(sc == m, taug, 9999)
        idx = jnp.min(cand, axis=2, keepdims=True)
        pick = jnp.logical_and(taug == idx, p < nsel)
        sel = jnp.logical_or(sel, pick)
        sc = jnp.where(pick, NEG, sc)
    mask_ref[...] = jnp.swapaxes(jnp.where(sel, 0.0, NEG), 0, 1)


def _run_a3(eq_b):
    return pl.pallas_call(
        _a3_body,
        grid=(1,),
        in_specs=[pl.BlockSpec((BS, SEQ, D), lambda i: (0, 0, 0))],
        out_specs=pl.BlockSpec((SEQ, BS, SEQ), lambda i: (0, 0, 0)),
        out_shape=jax.ShapeDtypeStruct((SEQ, BS, SEQ), jnp.float32),
    )(eq_b)


def _b_body(gi1_ref, qc8_ref, mask_ref, h1i_ref, h2i_ref,
            w1hh_ref, w2ih_ref, w2hh_ref, b1hh_ref, b2ih_ref, b2hh_ref,
            qw_ref, qb_ref, kb_ref, ww_ref, wb_ref,
            y_ref, h1_s, h2_s, hist_s):
    u = pl.program_id(0)
    ww = ww_ref[...]
    w1v = ww[:, :D]
    w2v = ww[:, D:]

    @pl.when(u == 0)
    def _init():
        h1_s[...] = h1i_ref[...]
        h2_s[...] = h2i_ref[...]
        hist_s[...] = jnp.zeros_like(hist_s)
        qv = w1v @ qw_ref[...]
        hist_s[:, SEQ:SEQ + 1, :] = jnp.broadcast_to(qv[None], (BS, 1, D))
        y_ref[...] = jnp.zeros_like(y_ref)

    @pl.when(u == 2)
    def _clear_row0():
        hist_s[:, 0:1, :] = jnp.zeros((BS, 1, D), jnp.float32)

    hist = hist_s[...]

    qb1 = jnp.sum(qb_ref[...] * w1v, axis=1, keepdims=True)
    kb2 = (jnp.sum(kb_ref[...] * w2v, axis=1, keepdims=True)
           + wb_ref[...][:, :1])
    qc8 = qc8_ref[0]
    d = lax.dot_general(qc8, hist, (((2,), (2,)), ((0,), (0,))))
    g = jax.nn.sigmoid(d[:, :5, :SEQ])
    qw1 = d[:, :5, SEQ:SEQ + 1]
    kw2 = d[:, 5:6, :SEQ]
    mt = mask_ref[0][:, None, :]
    w = qw1 + kw2 + mt + jnp.reshape(qb1 + kb2, (1, 1, 1))
    m = jnp.max(jnp.max(w, axis=2, keepdims=True), axis=1, keepdims=True)
    e = jnp.exp(w - m)
    num = jnp.sum(jnp.sum(e * g, axis=2, keepdims=True), axis=1)
    den = jnp.sum(jnp.sum(e, axis=2, keepdims=True), axis=1)
    yt = num / den

    @pl.when(u > 0)
    def _ywrite():
        col = jnp.where(u == 1, 0, u)
        lane = lax.broadcasted_iota(jnp.int32, (BS, SEQ), 1)
        y_ref[...] = jnp.where(lane == col, jnp.broadcast_to(yt, (BS, SEQ)),
                               y_ref[...])

    h1 = h1_s[...]
    h2 = h2_s[...]
    gi1 = gi1_ref[0]
    gh1 = h1 @ w1hh_ref[...] + b1hh_ref[...]
    r1 = jax.nn.sigmoid(gi1[:, :D] + gh1[:, :D])
    z1 = jax.nn.sigmoid(gi1[:, D:2 * D] + gh1[:, D:2 * D])
    n1 = jnp.tanh(gi1[:, 2 * D:] + r1 * gh1[:, 2 * D:])
    h1n = (1.0 - z1) * n1 + z1 * h1
    gi2 = h1n @ w2ih_ref[...] + b2ih_ref[...]
    gh2 = h2 @ w2hh_ref[...] + b2hh_ref[...]
    r2 = jax.nn.sigmoid(gi2[:, :D] + gh2[:, :D])
    z2 = jax.nn.sigmoid(gi2[:, D:2 * D] + gh2[:, D:2 * D])
    n2 = jnp.tanh(gi2[:, 2 * D:] + r2 * gh2[:, 2 * D:])
    g2 = (1.0 - z2) * n2 + z2 * h2

    @pl.when(u < SEQ - 1)
    def _state_upd():
        hist_s[:, pl.ds(u, 1), :] = g2[:, None, :]
        h1_s[...] = h1n

    @pl.when(jnp.logical_and(u >= 1, u < SEQ - 1))
    def _h2_upd():
        h2_s[...] = g2


def _run_b(gi1r, qc8r, mask_t, h1_init, h2_init,
           w1hh_t, w2ih_t, w2hh_t, b1hh, b2ih, b2hh,
           query_W, query_b, key_b, W_W, wb):
    return pl.pallas_call(
        _b_body,
        grid=(SEQ,),
        in_specs=[
            pl.BlockSpec((1, BS, 3 * D), lambda u: (jnp.minimum(u, SEQ - 2), 0, 0)),
            pl.BlockSpec((1, BS, 8, D), lambda u: (jnp.maximum(u - 1, 0), 0, 0, 0)),
            pl.BlockSpec((1, BS, SEQ), lambda u: (jnp.maximum(u - 1, 0), 0, 0)),
            pl.BlockSpec((BS, D), lambda t: (0, 0)),
            pl.BlockSpec((BS, D), lambda t: (0, 0)),
            pl.BlockSpec((D, 3 * D), lambda t: (0, 0)),
            pl.BlockSpec((D, 3 * D), lambda t: (0, 0)),
            pl.BlockSpec((D, 3 * D), lambda t: (0, 0)),
            pl.BlockSpec((1, 3 * D), lambda t: (0, 0)),
            pl.BlockSpec((1, 3 * D), lambda t: (0, 0)),
            pl.BlockSpec((1, 3 * D), lambda t: (0, 0)),
            pl.BlockSpec((D, D), lambda t: (0, 0)),
            pl.BlockSpec((1, D), lambda t: (0, 0)),
            pl.BlockSpec((1, D), lambda t: (0, 0)),
            pl.BlockSpec((1, 2 * D), lambda t: (0, 0)),
            pl.BlockSpec((1, D), lambda t: (0, 0)),
        ],
        out_specs=pl.BlockSpec((BS, SEQ), lambda t: (0, 0)),
        out_shape=jax.ShapeDtypeStruct((BS, SEQ), jnp.float32),
        scratch_shapes=[
            pltpu.VMEM((BS, D), jnp.float32),
            pltpu.VMEM((BS, D), jnp.float32),
            pltpu.VMEM((BS, HR, D), jnp.float32),
        ],
    )(gi1r, qc8r, mask_t, h1_init, h2_init,
      w1hh_t, w2ih_t, w2hh_t, b1hh, b2ih, b2hh,
      query_W, query_b, key_b, W_W, wb)


def kernel(question_seq, correct_seq, mask_seq, question_neighbors,
           concept_neighbors, q2c, embed_question, embed_concept,
           embed_correct, gru1_W_ih, gru1_W_hh, gru1_b_ih, gru1_b_hh,
           gru2_W_ih, gru2_W_hh, gru2_b_ih, gru2_b_hh,
           agg_W0, agg_b0, agg_W1, agg_b1, agg_last_W, agg_last_b,
           query_W, query_b, key_W, key_b, W_W, W_b, h1_init, h2_init):
    f32 = jnp.float32
    qseq_t = question_seq.astype(jnp.int32).T.reshape(-1)
    cn_idx2 = jnp.pad(concept_neighbors.astype(jnp.int32).reshape(-1),
                      (0, CN_PAD - NC * 16)).reshape(128, 128)
    int_tab = jnp.pad(
        jnp.concatenate([question_neighbors.astype(jnp.int32),
                         q2c.astype(jnp.int32)], axis=1),
        ((0, 0), (0, 108)))

    cn_rows, eq_rows, int_rows = _sc_gather(
        embed_question.astype(f32), int_tab, qseq_t, cn_idx2)

    cn3 = cn_rows[:NC * 16].reshape(NC, 16, D)
    ca = _run_a1(cn3, embed_concept.astype(f32), agg_W1.T,
                 agg_b1.reshape(1, D))

    int_shift = jnp.concatenate([int_rows[BS:], int_rows[:BS]], axis=0)
    eq_shift = jnp.concatenate([eq_rows[BS:], eq_rows[:BS]], axis=0)
    ms = mask_seq.astype(jnp.int32).T.reshape(NPOS, 1)
    rs = correct_seq.astype(jnp.int32).T.reshape(NPOS, 1)
    gi1, qc8 = _run_a2(
        int_rows, int_shift, eq_rows, eq_shift, ms, rs, ca,
        agg_W0.T, agg_last_W.T, agg_b0.reshape(1, D),
        agg_last_b.reshape(1, D),
        gru1_W_ih[:, :D].T, gru1_W_ih[:, D:].T,
        gru1_b_ih.reshape(1, 3 * D), embed_correct.astype(f32),
        key_W.astype(f32), W_W.reshape(1, 2 * D))

    eq_b = eq_rows.reshape(SEQ, BS, D).transpose(1, 0, 2)
    mask_t = _run_a3(eq_b)

    gi1r = gi1.reshape(SEQ, BS, 3 * D)
    qc8r = qc8.reshape(SEQ, BS, 8, D)
    y = _run_b(
        gi1r, qc8r, mask_t, h1_init.astype(f32), h2_init.astype(f32),
        gru1_W_hh.T, gru2_W_ih.T, gru2_W_hh.T,
        gru1_b_hh.reshape(1, 3 * D), gru2_b_ih.reshape(1, 3 * D),
        gru2_b_hh.reshape(1, 3 * D),
        query_W.astype(f32), query_b.reshape(1, D), key_b.reshape(1, D),
        W_W.reshape(1, 2 * D),
        jnp.broadcast_to(W_b.reshape(1, 1), (1, D)))
    return y


# speedup vs baseline: 1.3896x; 1.0149x over previous
import functools

import jax
import jax.numpy as jnp
from jax import lax
from jax.experimental import pallas as pl
from jax.experimental.pallas import tpu as pltpu
from jax.experimental.pallas import tpu_sc as plsc

BS = 32
SEQ = 64
D = 128
NQ = 50000
NC = 1000
NEG = -1e30
NPOS = BS * SEQ
CN_PAD = 16384
HR = 66


def _sc_gather(eq_tab, int_tab, qseq_idx, cn_idx2):
    mesh = plsc.VectorSubcoreMesh(core_axis_name="c", subcore_axis_name="s")

    @functools.partial(
        pl.kernel,
        mesh=mesh,
        out_type=[
            jax.ShapeDtypeStruct((CN_PAD, D), jnp.float32),
            jax.ShapeDtypeStruct((NPOS, D), jnp.float32),
            jax.ShapeDtypeStruct((NPOS, 128), jnp.int32),
        ],
        scratch_types=[
            pltpu.VMEM((4, 128), jnp.int32),
            pltpu.VMEM((64,), jnp.int32),
            pltpu.VMEM((512, D), jnp.float32),
            pltpu.VMEM((64, D), jnp.float32),
            pltpu.VMEM((64, 128), jnp.int32),
            pltpu.SemaphoreType.DMA,
        ],
    )
    def k(eq_hbm, int_hbm, qidx_hbm, cnidx_hbm, cn_out, eq_out, int_out,
          cnidx_v, qidx_v, cnrows_v, eqrows_v, introws_v, sem):
        wid = lax.axis_index("s") * 2 + lax.axis_index("c")
        pltpu.sync_copy(cnidx_hbm.at[pl.ds(wid * 4, 4)], cnidx_v)
        pltpu.sync_copy(qidx_hbm.at[pl.ds(wid * 64, 64)], qidx_v)
        cps = []
        for c in range(4):
            cps.append(pltpu.async_copy(
                eq_hbm.at[cnidx_v.at[c]],
                cnrows_v.at[pl.ds(c * 128, 128)], sem))
        cps.append(pltpu.async_copy(eq_hbm.at[qidx_v], eqrows_v, sem))
        cps.append(pltpu.async_copy(int_hbm.at[qidx_v], introws_v, sem))
        for cp in cps:
            cp.wait()
        pltpu.sync_copy(cnrows_v, cn_out.at[pl.ds(wid * 512, 512)])
        pltpu.sync_copy(eqrows_v, eq_out.at[pl.ds(wid * 64, 64)])
        pltpu.sync_copy(introws_v, int_out.at[pl.ds(wid * 64, 64)])

    return k(eq_tab, int_tab, qseq_idx, cn_idx2)


def _a1_body(cn3_ref, ec_ref, w1t_ref, b1_ref, out_ref):
    m = jnp.sum(cn3_ref[...], axis=1) * (1.0 / 16.0)
    ec = ec_ref[...]
    a1 = jnp.tanh((m + ec) @ w1t_ref[...] + b1_ref[...])
    out_ref[...] = jnp.concatenate([ec, a1], axis=1)


def _run_a1(cn3, ec, w1t, b1):
    return pl.pallas_call(
        _a1_body,
        grid=(5,),
        in_specs=[
            pl.BlockSpec((200, 16, D), lambda i: (i, 0, 0)),
            pl.BlockSpec((200, D), lambda i: (i, 0)),
            pl.BlockSpec((D, D), lambda i: (0, 0)),
            pl.BlockSpec((1, D), lambda i: (0, 0)),
        ],
        out_specs=pl.BlockSpec((200, 2 * D), lambda i: (i, 0)),
        out_shape=jax.ShapeDtypeStruct((NC, 2 * D), jnp.float32),
    )(cn3, ec, w1t, b1)


def _a2_body(int_ref, ints_ref, eq_ref, eqs_ref, ms_ref,
             rs_ref, ca_ref,
             w0t_ref, wlt_ref, b0_ref, bl_ref, wiat_ref, wibt_ref, bih_ref,
             er_ref, kw_ref, ww_ref, gi1_ref, qc8_ref):
    ints = int_ref[...]
    ints2 = ints_ref[...]
    iota_c = lax.broadcasted_iota(jnp.int32, (512, NC), 1)
    s = jnp.zeros((512, NC), jnp.float32)
    for j in range(16):
        s = s + (ints[:, j:j + 1] == iota_c).astype(jnp.float32)
    cam = (s @ ca_ref[...]) * (1.0 / 16.0)
    cmean = cam[:, :D]
    amean = cam[:, D:]
    eq = eq_ref[...]
    e0a = jnp.tanh((cmean + eq) @ w0t_ref[...] + b0_ref[...])
    e0b = jnp.tanh((amean + e0a) @ w0t_ref[...] + b0_ref[...])
    agg = jnp.tanh(e0b @ wlt_ref[...] + bl_ref[...])
    mf = ms_ref[...].astype(jnp.float32)
    emb_q = mf * agg + (1.0 - mf) * eq
    rf = rs_ref[...].astype(jnp.float32)
    er = er_ref[...]
    emb_r = rf * er[1:2, :] + (1.0 - rf) * er[0:1, :]
    gi1_ref[...] = emb_q @ wiat_ref[...] + emb_r @ wibt_ref[...] + bih_ref[...]
    qc8_ref[:, 0, :] = eqs_ref[...]
    ec = ca_ref[...][:, :D]
    for j in range(4):
        oh = (ints2[:, 16 + j:17 + j] == iota_c).astype(jnp.float32)
        qc8_ref[:, j + 1, :] = oh @ ec
    kv = ww_ref[...][:, D:] @ kw_ref[...]
    qc8_ref[:, 5, :] = jnp.broadcast_to(kv, (512, D))
    qc8_ref[:, 6, :] = jnp.zeros((512, D), jnp.float32)
    qc8_ref[:, 7, :] = jnp.zeros((512, D), jnp.float32)


def _run_a2(int_rows, int_shift, eq_rows, eq_shift, ms, rs, ca,
            w0t, wlt, b0, bl, wiat, wibt, bih, er, key_W, W_W):
    return pl.pallas_call(
        _a2_body,
        grid=(4,),
        in_specs=[
            pl.BlockSpec((512, 128), lambda k: (k, 0)),
            pl.BlockSpec((512, 128), lambda k: (k, 0)),
            pl.BlockSpec((512, D), lambda k: (k, 0)),
            pl.BlockSpec((512, D), lambda k: (k, 0)),
            pl.BlockSpec((512, 1), lambda k: (k, 0)),
            pl.BlockSpec((512, 1), lambda k: (k, 0)),
            pl.BlockSpec((NC, 2 * D), lambda k: (0, 0)),
            pl.BlockSpec((D, D), lambda k: (0, 0)),
            pl.BlockSpec((D, D), lambda k: (0, 0)),
            pl.BlockSpec((1, D), lambda k: (0, 0)),
            pl.BlockSpec((1, D), lambda k: (0, 0)),
            pl.BlockSpec((D, 3 * D), lambda k: (0, 0)),
            pl.BlockSpec((D, 3 * D), lambda k: (0, 0)),
            pl.BlockSpec((1, 3 * D), lambda k: (0, 0)),
            pl.BlockSpec((2, D), lambda k: (0, 0)),
            pl.BlockSpec((D, D), lambda k: (0, 0)),
            pl.BlockSpec((1, 2 * D), lambda k: (0, 0)),
        ],
        out_specs=[
            pl.BlockSpec((512, 3 * D), lambda k: (k, 0)),
            pl.BlockSpec((512, 8, D), lambda k: (k, 0, 0)),
        ],
        out_shape=[
            jax.ShapeDtypeStruct((NPOS, 3 * D), jnp.float32),
            jax.ShapeDtypeStruct((NPOS, 8, D), jnp.float32),
        ],
    )(int_rows, int_shift, eq_rows, eq_shift, ms, rs, ca,
      w0t, wlt, b0, bl, wiat, wibt, bih, er, key_W, W_W)


def _a3_body(eqb_ref, mask_ref):
    eq = eqb_ref[...]
    sh = jnp.concatenate([eq[:, 1:], eq[:, :1]], axis=1)
    smat = lax.dot_general(sh, eq, (((2,), (2,)), ((0,), (0,))))
    tg = lax.broadcasted_iota(jnp.int32, (BS, SEQ, SEQ), 1)
    taug = lax.broadcasted_iota(jnp.int32, (BS, SEQ, SEQ), 2)
    sc = jnp.where(taug < tg, smat, NEG)
    nsel = jnp.minimum(tg[:, :, :1], 10)
    sel = taug == tg
    for p in range(10):
        m = jnp.max(sc, axis=2, keepdims=True)
        cand = jnp.where(sc == m, taug, 9999)
        idx = jnp.min(cand, axis=2, keepdims=True)
        pick = jnp.logical_and(taug == idx, p < nsel)
        sel = jnp.logical_or(sel, pick)
        sc = jnp.where(pick, NEG, sc)
    mask_ref[...] = jnp.swapaxes(jnp.where(sel, 0.0, NEG), 0, 1)


def _run_a3(eq_b):
    return pl.pallas_call(
        _a3_body,
        grid=(1,),
        in_specs=[pl.BlockSpec((BS, SEQ, D), lambda i: (0, 0, 0))],
        out_specs=pl.BlockSpec((SEQ, BS, SEQ), lambda i: (0, 0, 0)),
        out_shape=jax.ShapeDtypeStruct((SEQ, BS, SEQ), jnp.float32),
    )(eq_b)


def _b_body(gi1_ref, qc8_ref, mask_ref, h1i_ref, h2i_ref,
            w1hh_ref, w2ih_ref, w2hh_ref, b1hh_ref, b2ih_ref, b2hh_ref,
            qw_ref, qb_ref, kb_ref, ww_ref, wb_ref,
            y_ref, h1_s, h2_s, hist_s):
    u = pl.program_id(0)
    ww = ww_ref[...]
    w1v = ww[:, :D]
    w2v = ww[:, D:]

    @pl.when(u == 0)
    def _init():
        h1_s[...] = h1i_ref[...]
        h2_s[...] = h2i_ref[...]
        hist_s[...] = jnp.zeros_like(hist_s)
        qv = w1v @ qw_ref[...]
        hist_s[:, SEQ:SEQ + 1, :] = jnp.broadcast_to(qv[None], (BS, 1, D))
        y_ref[...] = jnp.zeros_like(y_ref)

    @pl.when(u == 2)
    def _clear_row0():
        hist_s[:, 0:1, :] = jnp.zeros((BS, 1, D), jnp.float32)

    hist = hist_s[...]

    qb1 = jnp.sum(qb_ref[...] * w1v, axis=1, keepdims=True)
    kb2 = (jnp.sum(kb_ref[...] * w2v, axis=1, keepdims=True)
           + wb_ref[...][:, :1])
    qc8 = qc8_ref[0]
    d = lax.dot_general(qc8, hist, (((2,), (2,)), ((0,), (0,))))
    g = jax.nn.sigmoid(d[:, :5, :SEQ])
    qw1 = d[:, :5, SEQ:SEQ + 1]
    kw2 = d[:, 5:6, :SEQ]
    mt = mask_ref[0][:, None, :]
    w = qw1 + kw2 + mt + jnp.reshape(qb1 + kb2, (1, 1, 1))
    m = jnp.max(jnp.max(w, axis=2, keepdims=True), axis=1, keepdims=True)
    e = jnp.exp(w - m)
    num = jnp.sum(jnp.sum(e * g, axis=2, keepdims=True), axis=1)
    den = jnp.sum(jnp.sum(e, axis=2, keepdims=True), axis=1)
    yt = num / den

    @pl.when(u > 0)
    def _ywrite():
        col = jnp.where(u == 1, 0, u)
        lane = lax.broadcasted_iota(jnp.int32, (BS, SEQ), 1)
        y_ref[...] = jnp.where(lane == col, jnp.broadcast_to(yt, (BS, SEQ)),
                               y_ref[...])

    h1 = h1_s[...]
    h2 = h2_s[...]
    gi1 = gi1_ref[0]
    gh1 = h1 @ w1hh_ref[...] + b1hh_ref[...]
    r1 = jax.nn.sigmoid(gi1[:, :D] + gh1[:, :D])
    z1 = jax.nn.sigmoid(gi1[:, D:2 * D] + gh1[:, D:2 * D])
    n1 = jnp.tanh(gi1[:, 2 * D:] + r1 * gh1[:, 2 * D:])
    h1n = (1.0 - z1) * n1 + z1 * h1
    gi2 = h1n @ w2ih_ref[...] + b2ih_ref[...]
    gh2 = h2 @ w2hh_ref[...] + b2hh_ref[...]
    r2 = jax.nn.sigmoid(gi2[:, :D] + gh2[:, :D])
    z2 = jax.nn.sigmoid(gi2[:, D:2 * D] + gh2[:, D:2 * D])
    n2 = jnp.tanh(gi2[:, 2 * D:] + r2 * gh2[:, 2 * D:])
    g2 = (1.0 - z2) * n2 + z2 * h2

    @pl.when(u < SEQ - 1)
    def _state_upd():
        hist_s[:, pl.ds(u, 1), :] = g2[:, None, :]
        h1_s[...] = h1n

    @pl.when(jnp.logical_and(u >= 1, u < SEQ - 1))
    def _h2_upd():
        h2_s[...] = g2


def _run_b(gi1r, qc8r, mask_t, h1_init, h2_init,
           w1hh_t, w2ih_t, w2hh_t, b1hh, b2ih, b2hh,
           query_W, query_b, key_b, W_W, wb):
    return pl.pallas_call(
        _b_body,
        grid=(SEQ,),
        in_specs=[
            pl.BlockSpec((1, BS, 3 * D), lambda u: (jnp.minimum(u, SEQ - 2), 0, 0)),
            pl.BlockSpec((1, BS, 8, D), lambda u: (jnp.maximum(u - 1, 0), 0, 0, 0)),
            pl.BlockSpec((1, BS, SEQ), lambda u: (jnp.maximum(u - 1, 0), 0, 0)),
            pl.BlockSpec((BS, D), lambda t: (0, 0)),
            pl.BlockSpec((BS, D), lambda t: (0, 0)),
            pl.BlockSpec((D, 3 * D), lambda t: (0, 0)),
            pl.BlockSpec((D, 3 * D), lambda t: (0, 0)),
            pl.BlockSpec((D, 3 * D), lambda t: (0, 0)),
            pl.BlockSpec((1, 3 * D), lambda t: (0, 0)),
            pl.BlockSpec((1, 3 * D), lambda t: (0, 0)),
            pl.BlockSpec((1, 3 * D), lambda t: (0, 0)),
            pl.BlockSpec((D, D), lambda t: (0, 0)),
            pl.BlockSpec((1, D), lambda t: (0, 0)),
            pl.BlockSpec((1, D), lambda t: (0, 0)),
            pl.BlockSpec((1, 2 * D), lambda t: (0, 0)),
            pl.BlockSpec((1, D), lambda t: (0, 0)),
        ],
        out_specs=pl.BlockSpec((BS, SEQ), lambda t: (0, 0)),
        out_shape=jax.ShapeDtypeStruct((BS, SEQ), jnp.float32),
        scratch_shapes=[
            pltpu.VMEM((BS, D), jnp.float32),
            pltpu.VMEM((BS, D), jnp.float32),
            pltpu.VMEM((BS, HR, D), jnp.float32),
        ],
    )(gi1r, qc8r, mask_t, h1_init, h2_init,
      w1hh_t, w2ih_t, w2hh_t, b1hh, b2ih, b2hh,
      query_W, query_b, key_b, W_W, wb)


def kernel(question_seq, correct_seq, mask_seq, question_neighbors,
           concept_neighbors, q2c, embed_question, embed_concept,
           embed_correct, gru1_W_ih, gru1_W_hh, gru1_b_ih, gru1_b_hh,
           gru2_W_ih, gru2_W_hh, gru2_b_ih, gru2_b_hh,
           agg_W0, agg_b0, agg_W1, agg_b1, agg_last_W, agg_last_b,
           query_W, query_b, key_W, key_b, W_W, W_b, h1_init, h2_init):
    f32 = jnp.float32
    qseq_t = question_seq.astype(jnp.int32).T.reshape(-1)
    cn_idx2 = jnp.pad(concept_neighbors.astype(jnp.int32).reshape(-1),
                      (0, CN_PAD - NC * 16)).reshape(128, 128)
    int_tab = jnp.pad(
        jnp.concatenate([question_neighbors.astype(jnp.int32),
                         q2c.astype(jnp.int32)], axis=1),
        ((0, 0), (0, 108)))

    cn_rows, eq_rows, int_rows = _sc_gather(
        embed_question.astype(f32), int_tab, qseq_t, cn_idx2)

    cn3 = cn_rows[:NC * 16].reshape(NC, 16, D)
    ca = _run_a1(cn3, embed_concept.astype(f32), agg_W1.T,
                 agg_b1.reshape(1, D))

    int_shift = jnp.concatenate([int_rows[BS:], int_rows[:BS]], axis=0)
    eq_shift = jnp.concatenate([eq_rows[BS:], eq_rows[:BS]], axis=0)
    ms = mask_seq.astype(jnp.int32).T.reshape(NPOS, 1)
    rs = correct_seq.astype(jnp.int32).T.reshape(NPOS, 1)
    gi1, qc8 = _run_a2(
        int_rows, int_shift, eq_rows, eq_shift, ms, rs, ca,
        agg_W0.T, agg_last_W.T, agg_b0.reshape(1, D),
        agg_last_b.reshape(1, D),
        gru1_W_ih[:, :D].T, gru1_W_ih[:, D:].T,
        gru1_b_ih.reshape(1, 3 * D), embed_correct.astype(f32),
        key_W.astype(f32), W_W.reshape(1, 2 * D))

    eq_b = eq_rows.reshape(SEQ, BS, D).transpose(1, 0, 2)
    mask_t = _run_a3(eq_b)

    gi1r = gi1.reshape(SEQ, BS, 3 * D)
    qc8r = qc8.reshape(SEQ, BS, 8, D)
    y = _run_b(
        gi1r, qc8r, mask_t, h1_init.astype(f32), h2_init.astype(f32),
        gru1_W_hh.T, gru2_W_ih.T, gru2_W_hh.T,
        gru1_b_hh.reshape(1, 3 * D), gru2_b_ih.reshape(1, 3 * D),
        gru2_b_hh.reshape(1, 3 * D),
        query_W.astype(f32), query_b.reshape(1, D), key_b.reshape(1, D),
        W_W.reshape(1, 2 * D),
        jnp.broadcast_to(W_b.reshape(1, 1), (1, D)))
    return y
